# baseline (speedup 1.0000x reference)
import functools

import jax
import jax.numpy as jnp
from jax.experimental import pallas as pl
from jax.experimental.pallas import tpu as pltpu

N = 10000
D = 256
H = 256
BR = 400


def _tc1_body(s0_ref, w0_ref, w1b_ref, w2b_ref, b0_ref, x1_ref, p1_ref, p2_ref):
    s0 = s0_ref[...]
    x1_ref[...] = jnp.maximum(
        jnp.dot(s0, w0_ref[...], preferred_element_type=jnp.float32) + b0_ref[...], 0.0)
    p1_ref[...] = jnp.dot(s0, w1b_ref[...], preferred_element_type=jnp.float32)
    p2_ref[...] = jnp.dot(s0, w2b_ref[...], preferred_element_type=jnp.float32)


def _tc2_body(s1_ref, w1a_ref, p1_ref, b1_ref, x2_ref):
    x2_ref[...] = jnp.maximum(
        jnp.dot(s1_ref[...], w1a_ref[...], preferred_element_type=jnp.float32)
        + p1_ref[...] + b1_ref[...], 0.0)


def _tc3_body(s2_ref, w2a_ref, p2_ref, b2_ref, x1_ref, x2_ref, out_ref):
    x3 = jnp.maximum(
        jnp.dot(s2_ref[...], w2a_ref[...], preferred_element_type=jnp.float32)
        + p2_ref[...] + b2_ref[...], 0.0)
    out_ref[...] = jnp.maximum(jnp.maximum(x1_ref[...], x2_ref[...]), x3)


def _row_spec(w):
    return pl.BlockSpec((BR, w), lambda i: (i, 0))


def _full_spec(shape):
    return pl.BlockSpec(shape, lambda i: (0,) * len(shape))


def _tc1(s0, W0, b0, W1b, W2b):
    return pl.pallas_call(
        _tc1_body,
        grid=(N // BR,),
        in_specs=[_row_spec(D), _full_spec((D, H)), _full_spec((D, H)),
                  _full_spec((D, H)), _full_spec((1, H))],
        out_specs=[_row_spec(H), _row_spec(H), _row_spec(H)],
        out_shape=[jax.ShapeDtypeStruct((N, H), jnp.float32)] * 3,
    )(s0, W0, W1b, W2b, b0.reshape(1, H))


def _tc2(s1, W1a, p1, b1):
    return pl.pallas_call(
        _tc2_body,
        grid=(N // BR,),
        in_specs=[_row_spec(H), _full_spec((H, H)), _row_spec(H), _full_spec((1, H))],
        out_specs=_row_spec(H),
        out_shape=jax.ShapeDtypeStruct((N, H), jnp.float32),
    )(s1, W1a, p1, b1.reshape(1, H))


def _tc3(s2, W2a, p2, b2, x1, x2):
    return pl.pallas_call(
        _tc3_body,
        grid=(N // BR,),
        in_specs=[_row_spec(H), _full_spec((H, H)), _row_spec(H), _full_spec((1, H)),
                  _row_spec(H), _row_spec(H)],
        out_specs=_row_spec(H),
        out_shape=jax.ShapeDtypeStruct((N, H), jnp.float32),
    )(s2, W2a, p2, b2.reshape(1, H), x1, x2)


def kernel(x, edge_index, root_node_mask, W0, b0, W1, b1, W2, b2):
    src = edge_index[0]
    dst = edge_index[1]
    W1a, W1b = W1[:H], W1[H:]
    W2a, W2b = W2[:H], W2[H:]

    def seg(h):
        return jax.ops.segment_sum(h[src], dst, num_segments=N)

    s0 = seg(x)
    x1, p1, p2 = _tc1(s0, W0, b0, W1b, W2b)
    s1 = seg(x1)
    x2 = _tc2(s1, W1a, p1, b1)
    s2 = seg(x2)
    out = _tc3(s2, W2a, p2, b2, x1, x2)
    return jnp.where(root_node_mask[:, None], out, 0.0)


# speedup vs baseline: 3.0269x; 3.0269x over previous
import functools

import jax
import jax.numpy as jnp
from jax import lax
from jax.experimental import pallas as pl
from jax.experimental.pallas import tpu as pltpu
from jax.experimental.pallas import tpu_sc as plsc

N = 10000
D = 256
H = 256
FH = 128
BR = 400

NC, NS = 2, 16
CK = 128
NSLAB = 2
NCHUNKH = 40
TE = NSLAB * NCHUNKH * CK
EP = NS * TE
NACC = 10008
OSTEP = 624
OLEN = 640


_sc_mesh = plsc.VectorSubcoreMesh(
    core_axis_name="c", subcore_axis_name="s", num_cores=NC, num_subcores=NS)


@functools.partial(
    pl.kernel,
    out_type=[jax.ShapeDtypeStruct((N, FH), jnp.float32),
              jax.ShapeDtypeStruct((N, FH), jnp.float32)],
    mesh=_sc_mesh,
    scratch_types=[
        pltpu.VMEM((NCHUNKH, CK), jnp.int32),
        pltpu.VMEM((NCHUNKH, CK), jnp.int32),
        pltpu.VMEM((2, CK, FH), jnp.float32),
        pltpu.VMEM_SHARED((NACC, FH), jnp.float32),
        pltpu.SemaphoreType.DMA,
        pltpu.SemaphoreType.DMA,
    ],
)
def _sc_seg(hA, hB, zrows, srcp, dstp, outA, outB,
            src_v, dst_v, rows_v, acc, sem0, sem1):
    c = lax.axis_index("c")
    s = lax.axis_index("s")

    pltpu.sync_copy(zrows, acc.at[pl.ds(s * OSTEP, OLEN)])
    plsc.subcore_barrier()

    def run(h):
        for half in range(NSLAB):
            pltpu.sync_copy(srcp.at[s, half], src_v)
            pltpu.sync_copy(dstp.at[s, half], dst_v)

            pltpu.async_copy(h.at[src_v.at[0]], rows_v.at[0], sem0)

            def body(i, carry):
                g = 2 * i

                pltpu.async_copy(h.at[src_v.at[g + 1]], rows_v.at[1], sem1)

                pltpu.make_async_copy(h.at[src_v.at[0]], rows_v.at[0], sem0).wait()
                pltpu.sync_copy(rows_v.at[0], acc.at[dst_v.at[g]], add=True)

                @pl.when(g + 2 < NCHUNKH)
                def _():
                    pltpu.async_copy(h.at[src_v.at[g + 2]], rows_v.at[0], sem0)

                pltpu.make_async_copy(h.at[src_v.at[0]], rows_v.at[1], sem1).wait()
                pltpu.sync_copy(rows_v.at[1], acc.at[dst_v.at[g + 1]], add=True)

                return carry

            lax.fori_loop(0, NCHUNKH // 2, body, 0)

    @pl.when(c == 0)
    def _():
        run(hA)

    @pl.when(c == 1)
    def _():
        run(hB)

    plsc.subcore_barrier()

    @pl.when(c == 0)
    def _():
        pltpu.sync_copy(acc.at[pl.ds(s * OSTEP, OLEN)], outA.at[pl.ds(s * OSTEP, OLEN)])

    @pl.when(c == 1)
    def _():
        pltpu.sync_copy(acc.at[pl.ds(s * OSTEP, OLEN)], outB.at[pl.ds(s * OSTEP, OLEN)])


def _tc1_body(sA_ref, sB_ref, w0_ref, w1b_ref, w2b_ref, b0_ref,
              x1A_ref, x1B_ref, p1_ref, p2_ref):
    sA, sB = sA_ref[...], sB_ref[...]
    w0 = w0_ref[...]
    x1 = jnp.maximum(
        jnp.dot(sA, w0[:FH], preferred_element_type=jnp.float32)
        + jnp.dot(sB, w0[FH:], preferred_element_type=jnp.float32)
        + b0_ref[...], 0.0)
    x1A_ref[...] = x1[:, :FH]
    x1B_ref[...] = x1[:, FH:]
    w1b = w1b_ref[...]
    p1_ref[...] = (jnp.dot(sA, w1b[:FH], preferred_element_type=jnp.float32)
                   + jnp.dot(sB, w1b[FH:], preferred_element_type=jnp.float32))
    w2b = w2b_ref[...]
    p2_ref[...] = (jnp.dot(sA, w2b[:FH], preferred_element_type=jnp.float32)
                   + jnp.dot(sB, w2b[FH:], preferred_element_type=jnp.float32))


def _tc2_body(sA_ref, sB_ref, w1a_ref, p1_ref, b1_ref, x2A_ref, x2B_ref):
    w1a = w1a_ref[...]
    x2 = jnp.maximum(
        jnp.dot(sA_ref[...], w1a[:FH], preferred_element_type=jnp.float32)
        + jnp.dot(sB_ref[...], w1a[FH:], preferred_element_type=jnp.float32)
        + p1_ref[...] + b1_ref[...], 0.0)
    x2A_ref[...] = x2[:, :FH]
    x2B_ref[...] = x2[:, FH:]


def _tc3_body(sA_ref, sB_ref, w2a_ref, p2_ref, b2_ref,
              x1A_ref, x1B_ref, x2A_ref, x2B_ref, out_ref):
    w2a = w2a_ref[...]
    x3 = jnp.maximum(
        jnp.dot(sA_ref[...], w2a[:FH], preferred_element_type=jnp.float32)
        + jnp.dot(sB_ref[...], w2a[FH:], preferred_element_type=jnp.float32)
        + p2_ref[...] + b2_ref[...], 0.0)
    x1 = jnp.concatenate([x1A_ref[...], x1B_ref[...]], axis=1)
    x2 = jnp.concatenate([x2A_ref[...], x2B_ref[...]], axis=1)
    out_ref[...] = jnp.maximum(jnp.maximum(x1, x2), x3)


def _row_spec(w):
    return pl.BlockSpec((BR, w), lambda i: (i, 0))


def _full_spec(shape):
    return pl.BlockSpec(shape, lambda i: (0,) * len(shape))


def _tc1(sA, sB, W0, b0, W1b, W2b):
    return pl.pallas_call(
        _tc1_body,
        grid=(N // BR,),
        in_specs=[_row_spec(FH), _row_spec(FH), _full_spec((D, H)),
                  _full_spec((D, H)), _full_spec((D, H)), _full_spec((1, H))],
        out_specs=[_row_spec(FH), _row_spec(FH), _row_spec(H), _row_spec(H)],
        out_shape=[jax.ShapeDtypeStruct((N, FH), jnp.float32),
                   jax.ShapeDtypeStruct((N, FH), jnp.float32),
                   jax.ShapeDtypeStruct((N, H), jnp.float32),
                   jax.ShapeDtypeStruct((N, H), jnp.float32)],
    )(sA, sB, W0, W1b, W2b, b0.reshape(1, H))


def _tc2(sA, sB, W1a, p1, b1):
    return pl.pallas_call(
        _tc2_body,
        grid=(N // BR,),
        in_specs=[_row_spec(FH), _row_spec(FH), _full_spec((H, H)),
                  _row_spec(H), _full_spec((1, H))],
        out_specs=[_row_spec(FH), _row_spec(FH)],
        out_shape=[jax.ShapeDtypeStruct((N, FH), jnp.float32),
                   jax.ShapeDtypeStruct((N, FH), jnp.float32)],
    )(sA, sB, W1a, p1, b1.reshape(1, H))


def _tc3(sA, sB, W2a, p2, b2, x1A, x1B, x2A, x2B):
    return pl.pallas_call(
        _tc3_body,
        grid=(N // BR,),
        in_specs=[_row_spec(FH), _row_spec(FH), _full_spec((H, H)),
                  _row_spec(H), _full_spec((1, H)),
                  _row_spec(FH), _row_spec(FH), _row_spec(FH), _row_spec(FH)],
        out_specs=_row_spec(H),
        out_shape=jax.ShapeDtypeStruct((N, H), jnp.float32),
    )(sA, sB, W2a, p2, b2.reshape(1, H), x1A, x1B, x2A, x2B)


def kernel(x, edge_index, root_node_mask, W0, b0, W1, b1, W2, b2):
    src = edge_index[0]
    dst = edge_index[1]
    W1a, W1b = W1[:H], W1[H:]
    W2a, W2b = W2[:H], W2[H:]

    pad = EP - src.shape[0]
    srcp = jnp.concatenate([src, jnp.zeros((pad,), jnp.int32)]).reshape(
        NS, NSLAB, NCHUNKH, CK)
    dstp = jnp.concatenate([dst, jnp.full((pad,), N, jnp.int32)]).reshape(
        NS, NSLAB, NCHUNKH, CK)
    zrows = jnp.zeros((OLEN, FH), jnp.float32)

    xA = x[:, :FH]
    xB = x[:, FH:]

    s0A, s0B = _sc_seg(xA, xB, zrows, srcp, dstp)
    x1A, x1B, p1, p2 = _tc1(s0A, s0B, W0, b0, W1b, W2b)
    s1A, s1B = _sc_seg(x1A, x1B, zrows, srcp, dstp)
    x2A, x2B = _tc2(s1A, s1B, W1a, p1, b1)
    s2A, s2B = _sc_seg(x2A, x2B, zrows, srcp, dstp)
    out = _tc3(s2A, s2B, W2a, p2, b2, x1A, x1B, x2A, x2B)
    return jnp.where(root_node_mask[:, None], out, 0.0)
